# Initial kernel scaffold; baseline (speedup 1.0000x reference)
import jax
import jax.numpy as jnp
from jax.experimental import pallas as pl


def kernel(cls_score, label, class_freq, neg_class_freq, epoch):
    raise NotImplementedError("write your pallas kernel here")


# speedup vs baseline: 21.2714x; 21.2714x over previous
import math

import jax
import jax.numpy as jnp
from jax.experimental import pallas as pl
from jax.experimental.pallas import tpu as pltpu

B, C = 128, 8192
NEG_SCALE = 5.0
INIT_BIAS = 0.1
MAP_ALPHA, MAP_BETA, MAP_GAMMA = 10.0, 0.2, 0.1
FOCAL_GAMMA = 2.0
BALANCE_PARAM = 2.0
LOSS_WEIGHT = 1.0

CLEAN_RATE = 0.9
K_TOTAL = math.ceil(B * C * (1.0 - CLEAN_RATE))
P_K_MAX = math.ceil(K_TOTAL * 0.1)


def _main_kernel(tn_ref, score_ref, label_ref, cf_ref,
                 out_final_ref, out_loss_ref,
                 loss_ref, corr_ref, bits_ref):
    score = score_ref[...]
    lab_i = label_ref[...]
    cf = cf_ref[...]
    tn = tn_ref[0, 0]

    init_bias = -jnp.log(tn / cf - 1.0) * (INIT_BIAS / NEG_SCALE)
    freq_inv = 1.0 / cf
    labf = jnp.maximum(lab_i, 0).astype(jnp.float32)

    def loss_an(sb, lab):
        rr = jnp.sum(lab * freq_inv, axis=1, keepdims=True)
        pw = freq_inv / rr
        w = jax.nn.sigmoid(MAP_BETA * (pw - MAP_GAMMA)) + MAP_ALPHA
        logits = sb * (1.0 - lab) * NEG_SCALE + sb * lab
        w = w / NEG_SCALE * (1.0 - lab) + w * lab
        bce = (jnp.maximum(logits, 0.0) - logits * lab
               + jnp.log1p(jnp.exp(-jnp.abs(logits))))
        pt = jnp.exp(-bce)
        om = 1.0 - pt
        return (LOSS_WEIGHT * BALANCE_PARAM) * (om * om * (w * bce))

    s1 = score + init_bias
    loss = loss_an(s1, labf)
    corr = loss_an(s1 + init_bias, 1.0 - labf)
    loss_ref[...] = loss
    corr_ref[...] = corr
    bits_ref[...] = jax.lax.bitcast_convert_type(loss, jnp.int32)

    pos_f = jnp.sum(labf)
    p_k_f = jnp.minimum(jnp.float32(P_K_MAX), pos_f)
    n_k_f = jnp.float32(K_TOTAL) - p_k_f

    def body(i, carry):
        sel_n, sel_p = carry
        bit = 30 - i
        m = jax.lax.shift_left(jnp.int32(1), bit)
        cand_n = jax.lax.bitwise_or(sel_n, m)
        cand_p = jax.lax.bitwise_or(sel_p, m)
        b = bits_ref[...]
        neg = label_ref[...] == 0
        cnt_n = jnp.sum(jnp.where(neg & (b >= cand_n), 1.0, 0.0))
        cnt_p = jnp.sum(jnp.where((~neg) & (b >= cand_p), 1.0, 0.0))
        sel_n = jnp.where(cnt_n >= n_k_f, cand_n, sel_n)
        sel_p = jnp.where(cnt_p >= p_k_f, cand_p, sel_p)
        return sel_n, sel_p

    sel_n, sel_p = jax.lax.fori_loop(
        0, 31, body, (jnp.int32(0), jnp.int32(0)))
    thr_n = jax.lax.bitcast_convert_type(sel_n, jnp.float32)
    thr_p = jax.lax.bitcast_convert_type(sel_p, jnp.float32)

    loss2 = loss_ref[...]
    corr2 = corr_ref[...]
    neg = label_ref[...] == 0
    u0 = jnp.where(neg, loss2, 0.0)
    u1 = jnp.where(neg, 0.0, loss2)
    keep = (u0 < thr_n) & (u1 < thr_p)
    final = jnp.where(keep, loss2, corr2)
    out_final_ref[0, 0] = jnp.sum(final)
    out_loss_ref[0, 0] = jnp.sum(loss2)


def kernel(cls_score, label, class_freq, neg_class_freq, epoch=1):
    train_num = (class_freq[0] + neg_class_freq[0]).reshape(1, 1)
    cf = class_freq.reshape(1, C)
    sums = pl.pallas_call(
        _main_kernel,
        out_shape=[
            jax.ShapeDtypeStruct((1, 1), jnp.float32),
            jax.ShapeDtypeStruct((1, 1), jnp.float32),
        ],
        in_specs=[
            pl.BlockSpec(memory_space=pltpu.SMEM),
            pl.BlockSpec(memory_space=pltpu.VMEM),
            pl.BlockSpec(memory_space=pltpu.VMEM),
            pl.BlockSpec(memory_space=pltpu.VMEM),
        ],
        out_specs=[
            pl.BlockSpec(memory_space=pltpu.SMEM),
            pl.BlockSpec(memory_space=pltpu.SMEM),
        ],
        scratch_shapes=[
            pltpu.VMEM((B, C), jnp.float32),
            pltpu.VMEM((B, C), jnp.float32),
            pltpu.VMEM((B, C), jnp.int32),
        ],
        compiler_params=pltpu.CompilerParams(
            vmem_limit_bytes=100 * 1024 * 1024,
        ),
    )(train_num, cls_score, label, cf)
    inv_n = 1.0 / float(B * C)
    mean_final = sums[0][0, 0] * inv_n
    mean_loss = sums[1][0, 0] * inv_n
    return jnp.where(epoch == 0, mean_loss, mean_final)


# speedup vs baseline: 26.2210x; 1.2327x over previous
import math

import jax
import jax.numpy as jnp
from jax.experimental import pallas as pl
from jax.experimental.pallas import tpu as pltpu

B, C = 128, 8192
NEG_SCALE = 5.0
INIT_BIAS = 0.1
MAP_ALPHA, MAP_BETA, MAP_GAMMA = 10.0, 0.2, 0.1
FOCAL_GAMMA = 2.0
BALANCE_PARAM = 2.0
LOSS_WEIGHT = 1.0

CLEAN_RATE = 0.9
K_TOTAL = math.ceil(B * C * (1.0 - CLEAN_RATE))
P_K_MAX = math.ceil(K_TOTAL * 0.1)


def _main_kernel(tn_ref, score_ref, label_ref, cf_ref,
                 out_final_ref, out_loss_ref,
                 loss_ref, corr_ref, bits0_ref, bits1_ref):
    score = score_ref[...]
    lab_i = label_ref[...]
    cf = cf_ref[...]
    tn = tn_ref[0, 0]

    init_bias = -jnp.log(tn / cf - 1.0) * (INIT_BIAS / NEG_SCALE)
    freq_inv = 1.0 / cf
    labf = jnp.maximum(lab_i, 0).astype(jnp.float32)

    def loss_an(sb, lab):
        rr = jnp.sum(lab * freq_inv, axis=1, keepdims=True)
        pw = freq_inv / rr
        w = jax.nn.sigmoid(MAP_BETA * (pw - MAP_GAMMA)) + MAP_ALPHA
        logits = sb * (1.0 - lab) * NEG_SCALE + sb * lab
        w = w / NEG_SCALE * (1.0 - lab) + w * lab
        bce = (jnp.maximum(logits, 0.0) - logits * lab
               + jnp.log1p(jnp.exp(-jnp.abs(logits))))
        pt = jnp.exp(-bce)
        om = 1.0 - pt
        return (LOSS_WEIGHT * BALANCE_PARAM) * (om * om * (w * bce))

    s1 = score + init_bias
    loss = loss_an(s1, labf)
    corr = loss_an(s1 + init_bias, 1.0 - labf)
    loss_ref[...] = loss
    corr_ref[...] = corr
    bits = jax.lax.bitcast_convert_type(loss, jnp.int32)
    neg0 = lab_i == 0
    bits0_ref[...] = jnp.where(neg0, bits, -1)
    bits1_ref[...] = jnp.where(neg0, -1, bits)

    pos_f = jnp.sum(labf)
    p_k_f = jnp.minimum(jnp.float32(P_K_MAX), pos_f)
    n_k_f = jnp.float32(K_TOTAL) - p_k_f

    def body(i, carry):
        sel_n, sel_p = carry
        bit = 30 - i
        m = jax.lax.shift_left(jnp.int32(1), bit)
        cand_n = jax.lax.bitwise_or(sel_n, m)
        cand_p = jax.lax.bitwise_or(sel_p, m)
        cnt_n = jnp.sum(jnp.where(bits0_ref[...] >= cand_n, 1.0, 0.0))
        cnt_p = jnp.sum(jnp.where(bits1_ref[...] >= cand_p, 1.0, 0.0))
        sel_n = jnp.where(cnt_n >= n_k_f, cand_n, sel_n)
        sel_p = jnp.where(cnt_p >= p_k_f, cand_p, sel_p)
        return sel_n, sel_p

    sel_n, sel_p = jax.lax.fori_loop(
        0, 31, body, (jnp.int32(0), jnp.int32(0)))
    thr_n = jax.lax.bitcast_convert_type(sel_n, jnp.float32)
    thr_p = jax.lax.bitcast_convert_type(sel_p, jnp.float32)

    loss2 = loss_ref[...]
    corr2 = corr_ref[...]
    neg = label_ref[...] == 0
    u0 = jnp.where(neg, loss2, 0.0)
    u1 = jnp.where(neg, 0.0, loss2)
    keep = (u0 < thr_n) & (u1 < thr_p)
    final = jnp.where(keep, loss2, corr2)
    out_final_ref[0, 0] = jnp.sum(final)
    out_loss_ref[0, 0] = jnp.sum(loss2)


def kernel(cls_score, label, class_freq, neg_class_freq, epoch=1):
    train_num = (class_freq[0] + neg_class_freq[0]).reshape(1, 1)
    cf = class_freq.reshape(1, C)
    sums = pl.pallas_call(
        _main_kernel,
        out_shape=[
            jax.ShapeDtypeStruct((1, 1), jnp.float32),
            jax.ShapeDtypeStruct((1, 1), jnp.float32),
        ],
        in_specs=[
            pl.BlockSpec(memory_space=pltpu.SMEM),
            pl.BlockSpec(memory_space=pltpu.VMEM),
            pl.BlockSpec(memory_space=pltpu.VMEM),
            pl.BlockSpec(memory_space=pltpu.VMEM),
        ],
        out_specs=[
            pl.BlockSpec(memory_space=pltpu.SMEM),
            pl.BlockSpec(memory_space=pltpu.SMEM),
        ],
        scratch_shapes=[
            pltpu.VMEM((B, C), jnp.float32),
            pltpu.VMEM((B, C), jnp.float32),
            pltpu.VMEM((B, C), jnp.int32),
            pltpu.VMEM((B, C), jnp.int32),
        ],
        compiler_params=pltpu.CompilerParams(
            vmem_limit_bytes=100 * 1024 * 1024,
        ),
    )(train_num, cls_score, label, cf)
    inv_n = 1.0 / float(B * C)
    mean_final = sums[0][0, 0] * inv_n
    mean_loss = sums[1][0, 0] * inv_n
    return jnp.where(epoch == 0, mean_loss, mean_final)


# speedup vs baseline: 26.2617x; 1.0016x over previous
import math

import jax
import jax.numpy as jnp
from jax.experimental import pallas as pl
from jax.experimental.pallas import tpu as pltpu

B, C = 128, 8192
NEG_SCALE = 5.0
INIT_BIAS = 0.1
MAP_ALPHA, MAP_BETA, MAP_GAMMA = 10.0, 0.2, 0.1
FOCAL_GAMMA = 2.0
BALANCE_PARAM = 2.0
LOSS_WEIGHT = 1.0

CLEAN_RATE = 0.9
K_TOTAL = math.ceil(B * C * (1.0 - CLEAN_RATE))
P_K_MAX = math.ceil(K_TOTAL * 0.1)


def _main_kernel(tn_ref, score_ref, label_ref, cf_ref,
                 out_final_ref, out_loss_ref,
                 loss_ref, corr_ref, key_ref):
    score = score_ref[...]
    lab_i = label_ref[...]
    cf = cf_ref[...]
    tn = tn_ref[0, 0]

    init_bias = -jnp.log(tn / cf - 1.0) * (INIT_BIAS / NEG_SCALE)
    freq_inv = 1.0 / cf
    labf = jnp.maximum(lab_i, 0).astype(jnp.float32)

    def loss_an(sb, lab):
        rr = jnp.sum(lab * freq_inv, axis=1, keepdims=True)
        pw = freq_inv / rr
        w = jax.nn.sigmoid(MAP_BETA * (pw - MAP_GAMMA)) + MAP_ALPHA
        logits = sb * (1.0 - lab) * NEG_SCALE + sb * lab
        w = w / NEG_SCALE * (1.0 - lab) + w * lab
        bce = (jnp.maximum(logits, 0.0) - logits * lab
               + jnp.log1p(jnp.exp(-jnp.abs(logits))))
        pt = jnp.exp(-bce)
        om = 1.0 - pt
        return (LOSS_WEIGHT * BALANCE_PARAM) * (om * om * (w * bce))

    s1 = score + init_bias
    loss = loss_an(s1, labf)
    corr = loss_an(s1 + init_bias, 1.0 - labf)
    loss_ref[...] = loss
    corr_ref[...] = corr
    bits = jax.lax.bitcast_convert_type(loss, jnp.int32)
    neg0 = lab_i == 0
    key_ref[...] = jnp.where(neg0, bits, -bits)

    pos_f = jnp.sum(labf)
    p_k_f = jnp.minimum(jnp.float32(P_K_MAX), pos_f)
    n_k_f = jnp.float32(K_TOTAL) - p_k_f

    def body(i, carry):
        sel_n, sel_p = carry
        bit = 30 - i
        m = jax.lax.shift_left(jnp.int32(1), bit)
        cand_n = jax.lax.bitwise_or(sel_n, m)
        cand_p = jax.lax.bitwise_or(sel_p, m)
        key = key_ref[...]
        cnt_n = jnp.sum(jnp.where(key >= cand_n, 1.0, 0.0))
        cnt_p = jnp.sum(jnp.where(key <= -cand_p, 1.0, 0.0))
        sel_n = jnp.where(cnt_n >= n_k_f, cand_n, sel_n)
        sel_p = jnp.where(cnt_p >= p_k_f, cand_p, sel_p)
        return sel_n, sel_p

    sel_n, sel_p = jax.lax.fori_loop(
        0, 31, body, (jnp.int32(0), jnp.int32(0)))
    thr_n = jax.lax.bitcast_convert_type(sel_n, jnp.float32)
    thr_p = jax.lax.bitcast_convert_type(sel_p, jnp.float32)

    loss2 = loss_ref[...]
    corr2 = corr_ref[...]
    neg = label_ref[...] == 0
    u0 = jnp.where(neg, loss2, 0.0)
    u1 = jnp.where(neg, 0.0, loss2)
    keep = (u0 < thr_n) & (u1 < thr_p)
    final = jnp.where(keep, loss2, corr2)
    out_final_ref[0, 0] = jnp.sum(final)
    out_loss_ref[0, 0] = jnp.sum(loss2)


def kernel(cls_score, label, class_freq, neg_class_freq, epoch=1):
    train_num = (class_freq[0] + neg_class_freq[0]).reshape(1, 1)
    cf = class_freq.reshape(1, C)
    sums = pl.pallas_call(
        _main_kernel,
        out_shape=[
            jax.ShapeDtypeStruct((1, 1), jnp.float32),
            jax.ShapeDtypeStruct((1, 1), jnp.float32),
        ],
        in_specs=[
            pl.BlockSpec(memory_space=pltpu.SMEM),
            pl.BlockSpec(memory_space=pltpu.VMEM),
            pl.BlockSpec(memory_space=pltpu.VMEM),
            pl.BlockSpec(memory_space=pltpu.VMEM),
        ],
        out_specs=[
            pl.BlockSpec(memory_space=pltpu.SMEM),
            pl.BlockSpec(memory_space=pltpu.SMEM),
        ],
        scratch_shapes=[
            pltpu.VMEM((B, C), jnp.float32),
            pltpu.VMEM((B, C), jnp.float32),
            pltpu.VMEM((B, C), jnp.int32),
        ],
        compiler_params=pltpu.CompilerParams(
            vmem_limit_bytes=100 * 1024 * 1024,
        ),
    )(train_num, cls_score, label, cf)
    inv_n = 1.0 / float(B * C)
    mean_final = sums[0][0, 0] * inv_n
    mean_loss = sums[1][0, 0] * inv_n
    return jnp.where(epoch == 0, mean_loss, mean_final)
